# Initial kernel scaffold; baseline (speedup 1.0000x reference)
import jax
import jax.numpy as jnp
from jax.experimental import pallas as pl


def kernel(queries, keys):
    raise NotImplementedError("write your pallas kernel here")


# speedup vs baseline: 2.5779x; 2.5779x over previous
import functools

import jax
import jax.numpy as jnp
from jax.experimental import pallas as pl
from jax.experimental.pallas import tpu as pltpu

TOPK = 16
QB = 256
KB = 2048
RUNW = 128

_NEG_INF = float("-inf")
_BIG_IDX = 3.0e7


def _prenorm_body(k_ref, out_ref):
    k = k_ref[...]
    ss = jnp.sum(k * k, axis=1, keepdims=True)
    out_ref[...] = k / (jnp.sqrt(ss) + 1e-12)


def _knn_body(n_total, n_kb, q_ref, k_ref, vals_ref, idx_ref,
              qn_ref, runv_ref, runi_ref):
    kb = pl.program_id(1)

    @pl.when(kb == 0)
    def _init():
        q = q_ref[...]
        ss = jnp.sum(q * q, axis=1, keepdims=True)
        qn_ref[...] = q / (jnp.sqrt(ss) + 1e-12)
        runv_ref[...] = jnp.full((QB, RUNW), _NEG_INF, jnp.float32)
        runi_ref[...] = jnp.zeros((QB, RUNW), jnp.float32)

    qn = qn_ref[...]
    k = k_ref[...]
    s = jax.lax.dot_general(qn, k, (((1,), (1,)), ((), ())),
                            preferred_element_type=jnp.float32)
    gcol_i = jax.lax.broadcasted_iota(jnp.int32, (QB, KB), 1) + kb * KB
    s = jnp.where(gcol_i < n_total, s, _NEG_INF)
    gcol = gcol_i.astype(jnp.float32)

    v = jnp.concatenate([s, runv_ref[...]], axis=1)
    cidx = jnp.concatenate([gcol, runi_ref[...]], axis=1)

    ms, cs = [], []
    for _ in range(TOPK):
        m = jnp.max(v, axis=1, keepdims=True)
        c = jnp.min(jnp.where(v == m, cidx, _BIG_IDX), axis=1, keepdims=True)
        ms.append(m)
        cs.append(c)
        v = jnp.where(cidx == c, _NEG_INF, v)
    mv = jnp.concatenate(ms, axis=1)
    ci = jnp.concatenate(cs, axis=1)
    runv_ref[:, :TOPK] = mv
    runi_ref[:, :TOPK] = ci

    @pl.when(kb == n_kb - 1)
    def _out():
        vals_ref[...] = mv
        idx_ref[...] = ci.astype(jnp.int32)


@jax.jit
def kernel(queries, keys):
    q_n, d = queries.shape
    n = keys.shape[0]
    n_pad = pl.cdiv(n, KB) * KB
    n_kb = n_pad // KB
    n_qb = q_n // QB

    kpad = jnp.pad(keys, ((0, n_pad - n), (0, 0)))
    kn = pl.pallas_call(
        _prenorm_body,
        grid=(n_kb,),
        in_specs=[pl.BlockSpec((KB, d), lambda i: (i, 0))],
        out_specs=pl.BlockSpec((KB, d), lambda i: (i, 0)),
        out_shape=jax.ShapeDtypeStruct((n_pad, d), jnp.float32),
    )(kpad)

    vals, idx = pl.pallas_call(
        functools.partial(_knn_body, n, n_kb),
        grid=(n_qb, n_kb),
        in_specs=[
            pl.BlockSpec((QB, d), lambda qb, kb: (qb, 0)),
            pl.BlockSpec((KB, d), lambda qb, kb: (kb, 0)),
        ],
        out_specs=[
            pl.BlockSpec((QB, TOPK), lambda qb, kb: (qb, 0)),
            pl.BlockSpec((QB, TOPK), lambda qb, kb: (qb, 0)),
        ],
        out_shape=[
            jax.ShapeDtypeStruct((q_n, TOPK), jnp.float32),
            jax.ShapeDtypeStruct((q_n, TOPK), jnp.int32),
        ],
        scratch_shapes=[
            pltpu.VMEM((QB, d), jnp.float32),
            pltpu.VMEM((QB, RUNW), jnp.float32),
            pltpu.VMEM((QB, RUNW), jnp.float32),
        ],
    )(queries, kn)
    return vals, idx


# speedup vs baseline: 2.8371x; 1.1006x over previous
import functools

import jax
import jax.numpy as jnp
from jax.experimental import pallas as pl
from jax.experimental.pallas import tpu as pltpu

TOPK = 16
QB = 256
KB = 2048
SW = 512

_NEG_INF = float("-inf")
_BIG_IDX = 3.0e7


def _prenorm_body(k_ref, out_ref):
    k = k_ref[...]
    ss = jnp.sum(k * k, axis=1, keepdims=True)
    out_ref[...] = k / (jnp.sqrt(ss) + 1e-12)


def _knn_body(n_total, n_kb, q_ref, k_ref, vals_ref, idx_ref,
              qn_ref, rv_ref, ri_ref, vscr_ref):
    kb = pl.program_id(1)

    @pl.when(kb == 0)
    def _init():
        q = q_ref[...]
        ss = jnp.sum(q * q, axis=1, keepdims=True)
        qn_ref[...] = q / (jnp.sqrt(ss) + 1e-12)
        rv_ref[...] = jnp.full((QB, TOPK), _NEG_INF, jnp.float32)
        ri_ref[...] = jnp.zeros((QB, TOPK), jnp.float32)

    qn = qn_ref[...]
    k = k_ref[...]
    s = jax.lax.dot_general(qn, k, (((1,), (1,)), ((), ())),
                            preferred_element_type=jnp.float32)
    gcol_i = jax.lax.broadcasted_iota(jnp.int32, (QB, KB), 1) + kb * KB
    s = jnp.where(gcol_i < n_total, s, _NEG_INF)
    gcol = gcol_i.astype(jnp.float32)
    i16 = jax.lax.broadcasted_iota(jnp.int32, (QB, TOPK), 1)

    for t in range(KB // SW):
        sv = s[:, t * SW:(t + 1) * SW]
        gc = gcol[:, t * SW:(t + 1) * SW]
        m0 = jnp.max(sv, axis=1, keepdims=True)
        go0 = jnp.any(m0 > rv_ref[:, TOPK - 1:TOPK])

        @pl.when(go0)
        def _stage():
            vscr_ref[...] = sv

        def body(carry):
            _, m = carry
            v = vscr_ref[...]
            c = jnp.min(jnp.where(v == m, gc, _BIG_IDX), axis=1, keepdims=True)
            v2 = jnp.where(gc == c, _NEG_INF, v)
            vscr_ref[...] = v2
            rv = rv_ref[...]
            ri = ri_ref[...]
            pos = jnp.sum((rv >= m).astype(jnp.int32), axis=1, keepdims=True)
            sh_v = jnp.concatenate([rv[:, :1], rv[:, :TOPK - 1]], axis=1)
            sh_i = jnp.concatenate([ri[:, :1], ri[:, :TOPK - 1]], axis=1)
            nv = jnp.where(i16 < pos, rv, jnp.where(i16 == pos, m, sh_v))
            ni = jnp.where(i16 < pos, ri, jnp.where(i16 == pos, c, sh_i))
            rv_ref[...] = nv
            ri_ref[...] = ni
            m2 = jnp.max(v2, axis=1, keepdims=True)
            go2 = jnp.any(m2 > nv[:, TOPK - 1:TOPK])
            return go2, m2

        jax.lax.while_loop(lambda cy: cy[0], body, (go0, m0))

    @pl.when(kb == n_kb - 1)
    def _out():
        vals_ref[...] = rv_ref[...]
        idx_ref[...] = ri_ref[...].astype(jnp.int32)


@jax.jit
def kernel(queries, keys):
    q_n, d = queries.shape
    n = keys.shape[0]
    n_pad = pl.cdiv(n, KB) * KB
    n_kb = n_pad // KB
    n_qb = q_n // QB

    kpad = jnp.pad(keys, ((0, n_pad - n), (0, 0)))
    kn = pl.pallas_call(
        _prenorm_body,
        grid=(n_kb,),
        in_specs=[pl.BlockSpec((KB, d), lambda i: (i, 0))],
        out_specs=pl.BlockSpec((KB, d), lambda i: (i, 0)),
        out_shape=jax.ShapeDtypeStruct((n_pad, d), jnp.float32),
    )(kpad)

    vals, idx = pl.pallas_call(
        functools.partial(_knn_body, n, n_kb),
        grid=(n_qb, n_kb),
        in_specs=[
            pl.BlockSpec((QB, d), lambda qb, kb: (qb, 0)),
            pl.BlockSpec((KB, d), lambda qb, kb: (kb, 0)),
        ],
        out_specs=[
            pl.BlockSpec((QB, TOPK), lambda qb, kb: (qb, 0)),
            pl.BlockSpec((QB, TOPK), lambda qb, kb: (qb, 0)),
        ],
        out_shape=[
            jax.ShapeDtypeStruct((q_n, TOPK), jnp.float32),
            jax.ShapeDtypeStruct((q_n, TOPK), jnp.int32),
        ],
        scratch_shapes=[
            pltpu.VMEM((QB, d), jnp.float32),
            pltpu.VMEM((QB, TOPK), jnp.float32),
            pltpu.VMEM((QB, TOPK), jnp.float32),
            pltpu.VMEM((QB, SW), jnp.float32),
        ],
    )(queries, kn)
    return vals, idx
